# baseline (speedup 1.0000x reference)
import functools

import jax
import jax.numpy as jnp
import numpy as np
from jax import lax
from jax.experimental import pallas as pl
from jax.experimental.pallas import tpu as pltpu
from jax.experimental.pallas import tpu_sc as plsc

NC = 2
NS = 16
NW = NC * NS
L = 16
NB = 2048
CHUNK = 8192
_MININT = np.int32(-2147483648)


@functools.lru_cache(maxsize=None)
def _make_hist_kernel(n, shift, mshift, mmask):
    per_w = n // NW
    nchunks = per_w // CHUNK
    assert per_w * NW == n and nchunks * CHUNK == per_w and nchunks % 2 == 0

    mesh = plsc.VectorSubcoreMesh(
        core_axis_name="c", subcore_axis_name="s", num_cores=NC, num_subcores=NS
    )

    @functools.partial(
        pl.kernel,
        out_type=jax.ShapeDtypeStruct((NW, NB), jnp.int32),
        mesh=mesh,
        compiler_params=pltpu.CompilerParams(needs_layout_passes=False),
        scratch_types=[
            pltpu.VMEM((2 * CHUNK,), jnp.float32),
            pltpu.VMEM((2 * L * NB,), jnp.int32),
            pltpu.VMEM((NB,), jnp.int32),
            pltpu.VMEM((L,), jnp.int32),
            pltpu.SemaphoreType.DMA,
            pltpu.SemaphoreType.DMA,
        ],
    )
    def hist_kernel(x_hbm, pref_hbm, out_hbm, buf, hist, part, prefv, sem0, sem1):
        wid = lax.axis_index("s") * NC + lax.axis_index("c")
        base = wid * per_w

        pltpu.sync_copy(pref_hbm, prefv)
        pv = prefv[...]

        zero16 = jnp.zeros((L,), jnp.int32)

        def zbody(i, carry):
            hist[pl.ds(i * L, L)] = zero16
            return carry

        lax.fori_loop(0, (2 * L * NB) // L, zbody, 0)

        lane_base = lax.iota(jnp.int32, L) * NB
        ones16 = jnp.ones((L,), jnp.int32)
        sh = jnp.int32(shift)
        msh = jnp.int32(mshift)
        mmk = jnp.int32(mmask)

        def process(slot):
            boff = slot * CHUNK

            @plsc.parallel_loop(0, CHUNK // L, unroll=16)
            def _(i):
                v = buf[pl.ds(boff + i * L, L)]
                b = plsc.bitcast(v, jnp.int32)
                key = b ^ ((b >> 31) | _MININT)
                if shift > 0:
                    bucket = lax.shift_right_logical(key, sh)
                else:
                    bucket = key
                if shift + 11 < 32:
                    bucket = bucket & jnp.int32(NB - 1)
                copy_off = (i & 1) * (L * NB)
                idx = bucket + lane_base + copy_off
                if mmask != 0:
                    match = (lax.shift_right_logical(key, msh) & mmk) == pv
                    plsc.addupdate_scatter(hist, [idx], ones16, mask=match)
                else:
                    plsc.addupdate_scatter(hist, [idx], ones16)

        def dma_in(chunk, slot, sem):
            return pltpu.make_async_copy(
                x_hbm.at[pl.ds(base + chunk * CHUNK, CHUNK)],
                buf.at[pl.ds(slot * CHUNK, CHUNK)],
                sem,
            )

        dma_in(0, 0, sem0).start()

        def cbody(s, carry):
            c0 = 2 * s
            dma_in(c0 + 1, 1, sem1).start()
            dma_in(c0, 0, sem0).wait()
            process(0)

            @pl.when(c0 + 2 < nchunks)
            def _():
                dma_in(c0 + 2, 0, sem0).start()

            dma_in(c0 + 1, 1, sem1).wait()
            process(1)
            return carry

        lax.fori_loop(0, nchunks // 2, cbody, 0)

        def rbody(j, carry):
            acc = hist[pl.ds(j * L, L)]
            for bank in range(1, 2 * L):
                acc = acc + hist[pl.ds(bank * NB + j * L, L)]
            part[pl.ds(j * L, L)] = acc
            return carry

        lax.fori_loop(0, NB // L, rbody, 0)
        pltpu.sync_copy(part, out_hbm.at[wid])

    return hist_kernel


def _pick(hist_rows, k):
    h = jnp.sum(hist_rows, axis=0)
    c = jnp.cumsum(h)
    b = jnp.sum((c <= k).astype(jnp.int32))
    below = jnp.where(b > 0, c[jnp.maximum(b - 1, 0)], 0)
    return b, k - below


def _thr_body(m_ref, x_ref, o_ref):
    o_ref[...] = (x_ref[...] > m_ref[0, 0]).astype(jnp.float32)


@functools.lru_cache(maxsize=None)
def _make_threshold(rows, cols, block_rows):
    grid = rows // block_rows
    return pl.pallas_call(
        _thr_body,
        grid=(grid,),
        in_specs=[
            pl.BlockSpec((1, 1), lambda i: (0, 0)),
            pl.BlockSpec((block_rows, cols), lambda i: (i, 0)),
        ],
        out_specs=pl.BlockSpec((block_rows, cols), lambda i: (i, 0)),
        out_shape=jax.ShapeDtypeStruct((rows, cols), jnp.float32),
    )


def kernel(x):
    n = x.size
    rank = (n - 1) // 2
    xf = x.reshape(-1)

    splat = lambda v: jnp.broadcast_to(jnp.int32(v), (L,))

    h1 = _make_hist_kernel(n, 21, 0, 0)(xf, splat(0))
    b1, k1 = _pick(h1, jnp.int32(rank))

    h2 = _make_hist_kernel(n, 10, 21, 0x7FF)(xf, splat(0) + b1)
    b2, k2 = _pick(h2, k1)

    h3 = _make_hist_kernel(n, 0, 10, 0x3FFFFF)(xf, splat(0) + ((b1 << 11) | b2))
    b3, _ = _pick(h3, k2)

    mkey = (b1 << 21) | (b2 << 10) | b3
    mbits = jnp.where(mkey < 0, mkey ^ _MININT, ~mkey)
    m = lax.bitcast_convert_type(mbits, jnp.float32)

    cols = x.shape[-1]
    rows = n // cols
    out = _make_threshold(rows, cols, 512)(
        m.reshape(1, 1), x.reshape(rows, cols)
    )
    return out.reshape(x.shape)


# speedup vs baseline: 1.1377x; 1.1377x over previous
import functools

import jax
import jax.numpy as jnp
import numpy as np
from jax import lax
from jax.experimental import pallas as pl
from jax.experimental.pallas import tpu as pltpu
from jax.experimental.pallas import tpu_sc as plsc

NC = 2
NS = 16
NW = NC * NS
L = 16
NB = 2048
CHUNK = 8192
_MININT = np.int32(-2147483648)


@functools.lru_cache(maxsize=None)
def _make_hist_kernel(n, shift, mshift, mmask):
    per_w = n // NW
    nchunks = per_w // CHUNK
    assert per_w * NW == n and nchunks * CHUNK == per_w and nchunks % 2 == 0

    mesh = plsc.VectorSubcoreMesh(
        core_axis_name="c", subcore_axis_name="s", num_cores=NC, num_subcores=NS
    )

    @functools.partial(
        pl.kernel,
        out_type=jax.ShapeDtypeStruct((NW, NB), jnp.int32),
        mesh=mesh,
        compiler_params=pltpu.CompilerParams(needs_layout_passes=False),
        scratch_types=[
            pltpu.VMEM((2 * CHUNK,), jnp.float32),
            pltpu.VMEM((2 * L * NB,), jnp.int32),
            pltpu.VMEM((NB,), jnp.int32),
            pltpu.VMEM((L,), jnp.int32),
            pltpu.SemaphoreType.DMA,
            pltpu.SemaphoreType.DMA,
        ],
    )
    def hist_kernel(x_hbm, pref_hbm, out_hbm, buf, hist, part, prefv, sem0, sem1):
        wid = lax.axis_index("s") * NC + lax.axis_index("c")
        base = wid * per_w

        pltpu.sync_copy(pref_hbm, prefv)
        pv = prefv[...]

        zero16 = jnp.zeros((L,), jnp.int32)

        def zbody(i, carry):
            hist[pl.ds(i * L, L)] = zero16
            return carry

        lax.fori_loop(0, (2 * L * NB) // L, zbody, 0)

        lane_base = lax.iota(jnp.int32, L) * NB
        ones16 = jnp.ones((L,), jnp.int32)
        sh = jnp.int32(shift)
        msh = jnp.int32(mshift)
        mmk = jnp.int32(mmask)

        def process(slot):
            boff = slot * CHUNK

            @plsc.parallel_loop(0, CHUNK // L, unroll=8)
            def _(i):
                v = buf[pl.ds(boff + i * L, L)]
                b = plsc.bitcast(v, jnp.int32)
                key = b ^ ((b >> 31) | _MININT)
                if shift > 0:
                    bucket = lax.shift_right_logical(key, sh)
                else:
                    bucket = key
                if shift + 11 < 32:
                    bucket = bucket & jnp.int32(NB - 1)
                copy_off = (i & 1) * (L * NB)
                idx = bucket + lane_base + copy_off
                if mmask != 0:
                    match = lax.shift_right_logical(key, msh) == pv
                    plsc.addupdate_scatter(hist, [idx], ones16, mask=match)
                else:
                    plsc.addupdate_scatter(hist, [idx], ones16)

        def dma_in(chunk, slot, sem):
            return pltpu.make_async_copy(
                x_hbm.at[pl.ds(base + chunk * CHUNK, CHUNK)],
                buf.at[pl.ds(slot * CHUNK, CHUNK)],
                sem,
            )

        dma_in(0, 0, sem0).start()

        def cbody(s, carry):
            c0 = 2 * s
            dma_in(c0 + 1, 1, sem1).start()
            dma_in(c0, 0, sem0).wait()
            process(0)

            @pl.when(c0 + 2 < nchunks)
            def _():
                dma_in(c0 + 2, 0, sem0).start()

            dma_in(c0 + 1, 1, sem1).wait()
            process(1)
            return carry

        lax.fori_loop(0, nchunks // 2, cbody, 0)

        def rbody(j, carry):
            acc = hist[pl.ds(j * L, L)]
            for bank in range(1, 2 * L):
                acc = acc + hist[pl.ds(bank * NB + j * L, L)]
            part[pl.ds(j * L, L)] = acc
            return carry

        lax.fori_loop(0, NB // L, rbody, 0)
        pltpu.sync_copy(part, out_hbm.at[wid])

    return hist_kernel


CBUF = 32768


@functools.lru_cache(maxsize=None)
def _make_hist_compact_kernel(n):
    per_w = n // NW
    nchunks = per_w // CHUNK
    assert per_w * NW == n and nchunks * CHUNK == per_w and nchunks % 2 == 0

    mesh = plsc.VectorSubcoreMesh(
        core_axis_name="c", subcore_axis_name="s", num_cores=NC, num_subcores=NS
    )

    @functools.partial(
        pl.kernel,
        out_type=(
            jax.ShapeDtypeStruct((NW, NB), jnp.int32),
            jax.ShapeDtypeStruct((NW, L), jnp.int32),
            jax.ShapeDtypeStruct((NW * CBUF,), jnp.int32),
        ),
        mesh=mesh,
        compiler_params=pltpu.CompilerParams(needs_layout_passes=False),
        scratch_types=[
            pltpu.VMEM((2 * CHUNK,), jnp.float32),
            pltpu.VMEM((2 * L * NB,), jnp.int32),
            pltpu.VMEM((CBUF,), jnp.int32),
            pltpu.VMEM((NB,), jnp.int32),
            pltpu.VMEM((L,), jnp.int32),
            pltpu.VMEM((L,), jnp.int32),
            pltpu.SemaphoreType.DMA,
            pltpu.SemaphoreType.DMA,
        ],
    )
    def hist_kernel(
        x_hbm, pref_hbm, out_hbm, counts_hbm, cand_hbm,
        buf, hist, cbuf, part, prefv, cntv, sem0, sem1,
    ):
        wid = lax.axis_index("s") * NC + lax.axis_index("c")
        base = wid * per_w

        pltpu.sync_copy(pref_hbm, prefv)
        pv = prefv[...]

        zero16 = jnp.zeros((L,), jnp.int32)

        def zbody(i, carry):
            hist[pl.ds(i * L, L)] = zero16
            return carry

        lax.fori_loop(0, (2 * L * NB) // L, zbody, 0)

        lane_base = lax.iota(jnp.int32, L) * NB
        ones16 = jnp.ones((L,), jnp.int32)
        cap = jnp.int32(CBUF - L)

        def process(slot, fill_in):
            boff = slot * CHUNK

            @plsc.parallel_loop(0, CHUNK // L, carry=fill_in, unroll=8)
            def fill_out(i, fill):
                v = buf[pl.ds(boff + i * L, L)]
                b = plsc.bitcast(v, jnp.int32)
                key = b ^ ((b >> 31) | _MININT)
                bucket = lax.shift_right_logical(key, jnp.int32(10)) & jnp.int32(
                    NB - 1
                )
                match = lax.shift_right_logical(key, jnp.int32(21)) == pv
                copy_off = (i & 1) * (L * NB)
                plsc.addupdate_scatter(
                    hist, [bucket + lane_base + copy_off], ones16, mask=match
                )
                off = jnp.minimum(fill, cap)
                plsc.store_compressed(cbuf.at[pl.ds(off, L)], key, mask=match)
                cnt = plsc.all_reduce_population_count(match)
                return fill + cnt[0]

            return fill_out

        def dma_in(chunk, slot, sem):
            return pltpu.make_async_copy(
                x_hbm.at[pl.ds(base + chunk * CHUNK, CHUNK)],
                buf.at[pl.ds(slot * CHUNK, CHUNK)],
                sem,
            )

        dma_in(0, 0, sem0).start()

        def cbody(s, fill):
            c0 = 2 * s
            dma_in(c0 + 1, 1, sem1).start()
            dma_in(c0, 0, sem0).wait()
            fill = process(0, fill)

            @pl.when(c0 + 2 < nchunks)
            def _():
                dma_in(c0 + 2, 0, sem0).start()

            dma_in(c0 + 1, 1, sem1).wait()
            return process(1, fill)

        fill = lax.fori_loop(0, nchunks // 2, cbody, jnp.int32(0))

        def rbody(j, carry):
            acc = hist[pl.ds(j * L, L)]
            for bank in range(1, 2 * L):
                acc = acc + hist[pl.ds(bank * NB + j * L, L)]
            part[pl.ds(j * L, L)] = acc
            return carry

        lax.fori_loop(0, NB // L, rbody, 0)
        pltpu.sync_copy(part, out_hbm.at[wid])

        cntv[...] = jnp.broadcast_to(fill, (L,))
        pltpu.sync_copy(cntv, counts_hbm.at[wid])
        pltpu.sync_copy(cbuf, cand_hbm.at[pl.ds(wid * CBUF, CBUF)])

    return hist_kernel


@functools.lru_cache(maxsize=None)
def _make_cand_kernel(n):
    per_w = n // NW
    nchunks = per_w // CHUNK
    assert per_w * NW == n and nchunks * CHUNK == per_w

    mesh = plsc.VectorSubcoreMesh(
        core_axis_name="c", subcore_axis_name="s", num_cores=NC, num_subcores=NS
    )

    @functools.partial(
        pl.kernel,
        out_type=jax.ShapeDtypeStruct((NW, NB), jnp.int32),
        mesh=mesh,
        compiler_params=pltpu.CompilerParams(needs_layout_passes=False),
        scratch_types=[
            pltpu.VMEM((CHUNK,), jnp.int32),
            pltpu.VMEM((CHUNK,), jnp.float32),
            pltpu.VMEM((L * NB,), jnp.int32),
            pltpu.VMEM((NB,), jnp.int32),
            pltpu.VMEM((L,), jnp.int32),
            pltpu.VMEM((L,), jnp.int32),
        ],
    )
    def cand_kernel(
        cand_hbm, counts_hbm, x_hbm, pref_hbm, out_hbm,
        kbuf, fbuf, hist, part, prefv, cntv,
    ):
        wid = lax.axis_index("s") * NC + lax.axis_index("c")

        pltpu.sync_copy(pref_hbm, prefv)
        pltpu.sync_copy(counts_hbm.at[wid], cntv)
        pv = prefv[...]
        cnt_splat = cntv[...]
        cnt0 = cnt_splat[0]

        zero16 = jnp.zeros((L,), jnp.int32)

        def zbody(i, carry):
            hist[pl.ds(i * L, L)] = zero16
            return carry

        lax.fori_loop(0, (L * NB) // L, zbody, 0)

        lane_base = lax.iota(jnp.int32, L) * NB
        lane_iota = lax.iota(jnp.int32, L)
        ones16 = jnp.ones((L,), jnp.int32)

        @pl.when(cnt0 <= CBUF - L)
        def _fast():
            for c in range(CBUF // CHUNK):
                pltpu.sync_copy(
                    cand_hbm.at[pl.ds(wid * CBUF + c * CHUNK, CHUNK)], kbuf
                )

                @plsc.parallel_loop(0, CHUNK // L, unroll=8)
                def _(i):
                    key = kbuf[pl.ds(i * L, L)]
                    gidx = lane_iota + (i * L + c * CHUNK)
                    match = (
                        lax.shift_right_logical(key, jnp.int32(10)) == pv
                    ) & (gidx < cnt_splat)
                    bucket = key & jnp.int32(NB - 1)
                    plsc.addupdate_scatter(
                        hist, [bucket + lane_base], ones16, mask=match
                    )

        @pl.when(cnt0 > CBUF - L)
        def _slow():
            base = wid * per_w

            def chbody(ch, carry):
                pltpu.sync_copy(x_hbm.at[pl.ds(base + ch * CHUNK, CHUNK)], fbuf)

                @plsc.parallel_loop(0, CHUNK // L, unroll=8)
                def _(i):
                    v = fbuf[pl.ds(i * L, L)]
                    b = plsc.bitcast(v, jnp.int32)
                    key = b ^ ((b >> 31) | _MININT)
                    match = lax.shift_right_logical(key, jnp.int32(10)) == pv
                    bucket = key & jnp.int32(NB - 1)
                    plsc.addupdate_scatter(
                        hist, [bucket + lane_base], ones16, mask=match
                    )

                return carry

            lax.fori_loop(0, nchunks, chbody, 0)

        def rbody(j, carry):
            acc = hist[pl.ds(j * L, L)]
            for bank in range(1, L):
                acc = acc + hist[pl.ds(bank * NB + j * L, L)]
            part[pl.ds(j * L, L)] = acc
            return carry

        lax.fori_loop(0, NB // L, rbody, 0)
        pltpu.sync_copy(part, out_hbm.at[wid])

    return cand_kernel


def _pick(hist_rows, k):
    h = jnp.sum(hist_rows, axis=0)
    c = jnp.cumsum(h)
    b = jnp.sum((c <= k).astype(jnp.int32))
    below = jnp.where(b > 0, c[jnp.maximum(b - 1, 0)], 0)
    return b, k - below


def _thr_body(m_ref, x_ref, o_ref):
    o_ref[...] = (x_ref[...] > m_ref[0, 0]).astype(jnp.float32)


@functools.lru_cache(maxsize=None)
def _make_threshold(rows, cols, block_rows):
    grid = rows // block_rows
    return pl.pallas_call(
        _thr_body,
        grid=(grid,),
        in_specs=[
            pl.BlockSpec((1, 1), lambda i: (0, 0)),
            pl.BlockSpec((block_rows, cols), lambda i: (i, 0)),
        ],
        out_specs=pl.BlockSpec((block_rows, cols), lambda i: (i, 0)),
        out_shape=jax.ShapeDtypeStruct((rows, cols), jnp.float32),
    )


def kernel(x):
    n = x.size
    rank = (n - 1) // 2
    xf = x.reshape(-1)

    splat = lambda v: jnp.broadcast_to(jnp.int32(v), (L,))

    h1 = _make_hist_kernel(n, 21, 0, 0)(xf, splat(0))
    b1, k1 = _pick(h1, jnp.int32(rank))

    h2, counts, cand = _make_hist_compact_kernel(n)(xf, splat(0) + b1)
    b2, k2 = _pick(h2, k1)

    h3 = _make_cand_kernel(n)(
        cand, counts, xf, splat(0) + ((b1 << 11) | b2)
    )
    b3, _ = _pick(h3, k2)

    mkey = (b1 << 21) | (b2 << 10) | b3
    mbits = jnp.where(mkey < 0, mkey ^ _MININT, ~mkey)
    m = lax.bitcast_convert_type(mbits, jnp.float32)

    cols = x.shape[-1]
    rows = n // cols
    out = _make_threshold(rows, cols, 512)(
        m.reshape(1, 1), x.reshape(rows, cols)
    )
    return out.reshape(x.shape)


# speedup vs baseline: 1.3204x; 1.1605x over previous
import functools

import jax
import jax.numpy as jnp
import numpy as np
from jax import lax
from jax.experimental import pallas as pl
from jax.experimental.pallas import tpu as pltpu
from jax.experimental.pallas import tpu_sc as plsc

NC = 2
NS = 16
NW = NC * NS
L = 16
NB = 2048
CHUNK = 8192
_MININT = np.int32(-2147483648)


CROWS = 4


@functools.lru_cache(maxsize=None)
def _make_hist_kernel(rows, cols, shift, mshift, mmask):
    n = rows * cols
    rows_w = rows // NW
    nchunks = rows_w // CROWS
    assert rows_w * NW == rows and nchunks * CROWS == rows_w and nchunks % 2 == 0

    mesh = plsc.VectorSubcoreMesh(
        core_axis_name="c", subcore_axis_name="s", num_cores=NC, num_subcores=NS
    )

    @functools.partial(
        pl.kernel,
        out_type=jax.ShapeDtypeStruct((NW, NB), jnp.int32),
        mesh=mesh,
        compiler_params=pltpu.CompilerParams(needs_layout_passes=False),
        scratch_types=[
            pltpu.VMEM((2 * CROWS, cols), jnp.float32),
            pltpu.VMEM((2 * L * NB,), jnp.int32),
            pltpu.VMEM((NB,), jnp.int32),
            pltpu.VMEM((L,), jnp.int32),
            pltpu.SemaphoreType.DMA,
            pltpu.SemaphoreType.DMA,
        ],
    )
    def hist_kernel(x_hbm, pref_hbm, out_hbm, buf, hist, part, prefv, sem0, sem1):
        wid = lax.axis_index("s") * NC + lax.axis_index("c")
        base_row = wid * rows_w

        pltpu.sync_copy(pref_hbm, prefv)
        pv = prefv[...]

        zero16 = jnp.zeros((L,), jnp.int32)

        def zbody(i, carry):
            hist[pl.ds(i * L, L)] = zero16
            return carry

        lax.fori_loop(0, (2 * L * NB) // L, zbody, 0)

        lane_base = lax.iota(jnp.int32, L) * NB
        ones16 = jnp.ones((L,), jnp.int32)
        sh = jnp.int32(shift)
        msh = jnp.int32(mshift)
        mmk = jnp.int32(mmask)

        vec_per_row = cols // L

        def process(slot):
            @plsc.parallel_loop(0, CROWS * vec_per_row, unroll=8)
            def _(i):
                row = slot * CROWS + (i // vec_per_row)
                col = (i % vec_per_row) * L
                v = buf[row, pl.ds(col, L)]
                b = plsc.bitcast(v, jnp.int32)
                key = b ^ ((b >> 31) | _MININT)
                if shift > 0:
                    bucket = lax.shift_right_logical(key, sh)
                else:
                    bucket = key
                if shift + 11 < 32:
                    bucket = bucket & jnp.int32(NB - 1)
                copy_off = (i & 1) * (L * NB)
                idx = bucket + lane_base + copy_off
                if mmask != 0:
                    match = lax.shift_right_logical(key, msh) == pv
                    plsc.addupdate_scatter(hist, [idx], ones16, mask=match)
                else:
                    plsc.addupdate_scatter(hist, [idx], ones16)

        def dma_in(chunk, slot, sem):
            return pltpu.make_async_copy(
                x_hbm.at[pl.ds(base_row + chunk * CROWS, CROWS), :],
                buf.at[pl.ds(slot * CROWS, CROWS), :],
                sem,
            )

        dma_in(0, 0, sem0).start()

        def cbody(s, carry):
            c0 = 2 * s
            dma_in(c0 + 1, 1, sem1).start()
            dma_in(c0, 0, sem0).wait()
            process(0)

            @pl.when(c0 + 2 < nchunks)
            def _():
                dma_in(c0 + 2, 0, sem0).start()

            dma_in(c0 + 1, 1, sem1).wait()
            process(1)
            return carry

        lax.fori_loop(0, nchunks // 2, cbody, 0)

        def rbody(j, carry):
            acc = hist[pl.ds(j * L, L)]
            for bank in range(1, 2 * L):
                acc = acc + hist[pl.ds(bank * NB + j * L, L)]
            part[pl.ds(j * L, L)] = acc
            return carry

        lax.fori_loop(0, NB // L, rbody, 0)
        pltpu.sync_copy(part, out_hbm.at[wid])

    return hist_kernel


CBUF = 32768


@functools.lru_cache(maxsize=None)
def _make_hist_compact_kernel(rows, cols):
    rows_w = rows // NW
    nchunks = rows_w // CROWS
    assert rows_w * NW == rows and nchunks * CROWS == rows_w and nchunks % 2 == 0

    mesh = plsc.VectorSubcoreMesh(
        core_axis_name="c", subcore_axis_name="s", num_cores=NC, num_subcores=NS
    )

    @functools.partial(
        pl.kernel,
        out_type=(
            jax.ShapeDtypeStruct((NW, NB), jnp.int32),
            jax.ShapeDtypeStruct((NW, L), jnp.int32),
            jax.ShapeDtypeStruct((NW * CBUF,), jnp.int32),
        ),
        mesh=mesh,
        compiler_params=pltpu.CompilerParams(needs_layout_passes=False),
        scratch_types=[
            pltpu.VMEM((2 * CROWS, cols), jnp.float32),
            pltpu.VMEM((2 * L * NB,), jnp.int32),
            pltpu.VMEM((CBUF,), jnp.int32),
            pltpu.VMEM((NB,), jnp.int32),
            pltpu.VMEM((L,), jnp.int32),
            pltpu.VMEM((L,), jnp.int32),
            pltpu.SemaphoreType.DMA,
            pltpu.SemaphoreType.DMA,
        ],
    )
    def hist_kernel(
        x_hbm, pref_hbm, out_hbm, counts_hbm, cand_hbm,
        buf, hist, cbuf, part, prefv, cntv, sem0, sem1,
    ):
        wid = lax.axis_index("s") * NC + lax.axis_index("c")
        base_row = wid * rows_w

        pltpu.sync_copy(pref_hbm, prefv)
        pv = prefv[...]

        zero16 = jnp.zeros((L,), jnp.int32)

        def zbody(i, carry):
            hist[pl.ds(i * L, L)] = zero16
            return carry

        lax.fori_loop(0, (2 * L * NB) // L, zbody, 0)

        lane_base = lax.iota(jnp.int32, L) * NB
        ones16 = jnp.ones((L,), jnp.int32)
        cap = jnp.int32(CBUF - L)

        vec_per_row = cols // L

        def process(slot, fill_in):
            @plsc.parallel_loop(
                0, CROWS * vec_per_row, carry=fill_in, unroll=8
            )
            def fill_out(i, fill):
                row = slot * CROWS + (i // vec_per_row)
                col = (i % vec_per_row) * L
                v = buf[row, pl.ds(col, L)]
                b = plsc.bitcast(v, jnp.int32)
                key = b ^ ((b >> 31) | _MININT)
                bucket = lax.shift_right_logical(
                    key, jnp.int32(10)
                ) & jnp.int32(NB - 1)
                match = lax.shift_right_logical(key, jnp.int32(21)) == pv
                copy_off = (i & 1) * (L * NB)
                plsc.addupdate_scatter(
                    hist, [bucket + lane_base + copy_off], ones16, mask=match
                )
                off = jnp.minimum(fill, cap)
                plsc.store_compressed(cbuf.at[pl.ds(off, L)], key, mask=match)
                cnt = plsc.all_reduce_population_count(match)
                return fill + cnt[0]

            return fill_out

        def dma_in(chunk, slot, sem):
            return pltpu.make_async_copy(
                x_hbm.at[pl.ds(base_row + chunk * CROWS, CROWS), :],
                buf.at[pl.ds(slot * CROWS, CROWS), :],
                sem,
            )

        dma_in(0, 0, sem0).start()

        def cbody(s, fill):
            c0 = 2 * s
            dma_in(c0 + 1, 1, sem1).start()
            dma_in(c0, 0, sem0).wait()
            fill = process(0, fill)

            @pl.when(c0 + 2 < nchunks)
            def _():
                dma_in(c0 + 2, 0, sem0).start()

            dma_in(c0 + 1, 1, sem1).wait()
            return process(1, fill)

        fill = lax.fori_loop(0, nchunks // 2, cbody, jnp.int32(0))

        def rbody(j, carry):
            acc = hist[pl.ds(j * L, L)]
            for bank in range(1, 2 * L):
                acc = acc + hist[pl.ds(bank * NB + j * L, L)]
            part[pl.ds(j * L, L)] = acc
            return carry

        lax.fori_loop(0, NB // L, rbody, 0)
        pltpu.sync_copy(part, out_hbm.at[wid])

        cntv[...] = jnp.broadcast_to(fill, (L,))
        pltpu.sync_copy(cntv, counts_hbm.at[wid])
        pltpu.sync_copy(cbuf, cand_hbm.at[pl.ds(wid * CBUF, CBUF)])

    return hist_kernel


@functools.lru_cache(maxsize=None)
def _make_cand_kernel(rows, cols):
    rows_w = rows // NW
    nchunks = rows_w // CROWS
    assert rows_w * NW == rows and nchunks * CROWS == rows_w

    mesh = plsc.VectorSubcoreMesh(
        core_axis_name="c", subcore_axis_name="s", num_cores=NC, num_subcores=NS
    )

    @functools.partial(
        pl.kernel,
        out_type=jax.ShapeDtypeStruct((NW, NB), jnp.int32),
        mesh=mesh,
        compiler_params=pltpu.CompilerParams(needs_layout_passes=False),
        scratch_types=[
            pltpu.VMEM((CHUNK,), jnp.int32),
            pltpu.VMEM((CROWS, cols), jnp.float32),
            pltpu.VMEM((L * NB,), jnp.int32),
            pltpu.VMEM((NB,), jnp.int32),
            pltpu.VMEM((L,), jnp.int32),
            pltpu.VMEM((L,), jnp.int32),
        ],
    )
    def cand_kernel(
        cand_hbm, counts_hbm, x_hbm, pref_hbm, out_hbm,
        kbuf, fbuf, hist, part, prefv, cntv,
    ):
        wid = lax.axis_index("s") * NC + lax.axis_index("c")

        pltpu.sync_copy(pref_hbm, prefv)
        pltpu.sync_copy(counts_hbm.at[wid], cntv)
        pv = prefv[...]
        cnt_splat = cntv[...]
        cnt0 = cnt_splat[0]

        zero16 = jnp.zeros((L,), jnp.int32)

        def zbody(i, carry):
            hist[pl.ds(i * L, L)] = zero16
            return carry

        lax.fori_loop(0, (L * NB) // L, zbody, 0)

        lane_base = lax.iota(jnp.int32, L) * NB
        lane_iota = lax.iota(jnp.int32, L)
        ones16 = jnp.ones((L,), jnp.int32)

        @pl.when(cnt0 <= CBUF - L)
        def _fast():
            for c in range(CBUF // CHUNK):
                pltpu.sync_copy(
                    cand_hbm.at[pl.ds(wid * CBUF + c * CHUNK, CHUNK)], kbuf
                )

                @plsc.parallel_loop(0, CHUNK // L, unroll=8)
                def _(i):
                    key = kbuf[pl.ds(i * L, L)]
                    gidx = lane_iota + (i * L + c * CHUNK)
                    match = (
                        lax.shift_right_logical(key, jnp.int32(10)) == pv
                    ) & (gidx < cnt_splat)
                    bucket = key & jnp.int32(NB - 1)
                    plsc.addupdate_scatter(
                        hist, [bucket + lane_base], ones16, mask=match
                    )

        @pl.when(cnt0 > CBUF - L)
        def _slow():
            base_row = wid * rows_w

            def chbody(ch, carry):
                pltpu.sync_copy(
                    x_hbm.at[pl.ds(base_row + ch * CROWS, CROWS), :], fbuf
                )

                @plsc.parallel_loop(0, cols // L, unroll=8)
                def _(i):
                    for r in range(CROWS):
                        v = fbuf[r, pl.ds(i * L, L)]
                        b = plsc.bitcast(v, jnp.int32)
                        key = b ^ ((b >> 31) | _MININT)
                        match = (
                            lax.shift_right_logical(key, jnp.int32(10)) == pv
                        )
                        bucket = key & jnp.int32(NB - 1)
                        plsc.addupdate_scatter(
                            hist, [bucket + lane_base], ones16, mask=match
                        )

                return carry

            lax.fori_loop(0, nchunks, chbody, 0)

        def rbody(j, carry):
            acc = hist[pl.ds(j * L, L)]
            for bank in range(1, L):
                acc = acc + hist[pl.ds(bank * NB + j * L, L)]
            part[pl.ds(j * L, L)] = acc
            return carry

        lax.fori_loop(0, NB // L, rbody, 0)
        pltpu.sync_copy(part, out_hbm.at[wid])

    return cand_kernel


def _pick(hist_rows, k):
    h = jnp.sum(hist_rows, axis=0)
    c = jnp.cumsum(h)
    b = jnp.sum((c <= k).astype(jnp.int32))
    below = jnp.where(b > 0, c[jnp.maximum(b - 1, 0)], 0)
    return b, k - below


def _thr_body(m_ref, x_ref, o_ref):
    o_ref[...] = (x_ref[...] > m_ref[0, 0]).astype(jnp.float32)


@functools.lru_cache(maxsize=None)
def _make_threshold(rows, cols, block_rows):
    grid = rows // block_rows
    return pl.pallas_call(
        _thr_body,
        grid=(grid,),
        in_specs=[
            pl.BlockSpec((1, 1), lambda i: (0, 0)),
            pl.BlockSpec((block_rows, cols), lambda i: (i, 0)),
        ],
        out_specs=pl.BlockSpec((block_rows, cols), lambda i: (i, 0)),
        out_shape=jax.ShapeDtypeStruct((rows, cols), jnp.float32),
    )


def kernel(x):
    n = x.size
    rank = (n - 1) // 2
    cols = x.shape[-1]
    rows = n // cols
    xf = x.reshape(rows, cols)

    splat = lambda v: jnp.broadcast_to(jnp.int32(v), (L,))

    h1 = _make_hist_kernel(rows, cols, 21, 0, 0)(xf, splat(0))
    b1, k1 = _pick(h1, jnp.int32(rank))

    h2, counts, cand = _make_hist_compact_kernel(rows, cols)(xf, splat(0) + b1)
    b2, k2 = _pick(h2, k1)

    h3 = _make_cand_kernel(rows, cols)(
        cand, counts, xf, splat(0) + ((b1 << 11) | b2)
    )
    b3, _ = _pick(h3, k2)

    mkey = (b1 << 21) | (b2 << 10) | b3
    mbits = jnp.where(mkey < 0, mkey ^ _MININT, ~mkey)
    m = lax.bitcast_convert_type(mbits, jnp.float32)

    out = _make_threshold(rows, cols, 512)(m.reshape(1, 1), xf)
    return out.reshape(x.shape)
